# Initial kernel scaffold; baseline (speedup 1.0000x reference)
import jax
import jax.numpy as jnp
from jax.experimental import pallas as pl


def kernel(x, edge_index, edge_attr, pos, batch, params):
    raise NotImplementedError("write your pallas kernel here")


# speedup vs baseline: 1.1886x; 1.1886x over previous
import functools
import math

import jax
import jax.numpy as jnp
from jax.experimental import pallas as pl
from jax.experimental.pallas import tpu as pltpu

N_BLK = 625


def _mm_body(a_ref, b_ref, o_ref):
    o_ref[...] = jnp.dot(a_ref[...], b_ref[...],
                         preferred_element_type=jnp.float32)


def _rup(v, q):
    return (v + q - 1) // q * q


def _matmul(a, b):
    n, k = a.shape
    m = b.shape[1]
    bn = 1024
    np_, kp, mp = _rup(n, bn), _rup(k, 128), _rup(m, 128)
    ap = jnp.pad(a, ((0, np_ - n), (0, kp - k)))
    bp = jnp.pad(b, ((0, kp - k), (0, mp - m)))
    out = pl.pallas_call(
        _mm_body,
        grid=(np_ // bn,),
        in_specs=[pl.BlockSpec((bn, kp), lambda i: (i, 0)),
                  pl.BlockSpec((kp, mp), lambda i: (0, 0))],
        out_specs=pl.BlockSpec((bn, mp), lambda i: (i, 0)),
        out_shape=jax.ShapeDtypeStruct((np_, mp), jnp.float32),
    )(ap, bp)
    return out[:n, :m]


def _basis(ea, K):
    v = jnp.clip(ea, 0.0, 1.0) * (K - 1)
    lo = jnp.clip(jnp.floor(v), 0.0, float(max(K - 2, 0)))
    frac = v - lo
    lo_i = lo.astype(jnp.int32)
    ws, idxs = [], []
    for bits in range(4):
        w = jnp.ones((ea.shape[0],), ea.dtype)
        idx = jnp.zeros((ea.shape[0],), jnp.int32)
        mult = 1
        for d in range(2):
            b = (bits >> d) & 1
            w = w * (frac[:, d] if b else (1.0 - frac[:, d]))
            idx = idx + jnp.minimum(lo_i[:, d] + b, K - 1) * mult
            mult *= K
        ws.append(w)
        idxs.append(idx)
    return jnp.stack(ws, 1), jnp.stack(idxs, 1)


def _conv(x, src, dst, valid, bw, bidx, p, K):
    N = x.shape[0]
    W = p["weight"]
    Ktot, Cin, Cout = W.shape
    Wf = jnp.transpose(W, (1, 0, 2)).reshape(Cin, Ktot * Cout)
    if Cin == 1:
        xW = (x * Wf).reshape(N * Ktot, Cout)
    else:
        xW = _matmul(x, Wf).reshape(N * Ktot, Cout)
    if Ktot == 1:
        gidx = src[:, None]
        we = valid[:, None]
    else:
        gidx = src[:, None] * Ktot + bidx
        we = bw * valid[:, None]
    rows = xW[gidx]
    msg = jnp.sum(rows * we[..., None], axis=1)
    agg = jax.ops.segment_sum(msg, dst, num_segments=N)
    cnt = jax.ops.segment_sum(valid, dst, num_segments=N)
    out = agg / jnp.maximum(cnt, 1.0)[:, None]
    if Cin == 1:
        root_term = x * p["root"]
    else:
        root_term = _matmul(x, p["root"])
    return out + root_term + p["bias"]


def _topk_pool(x, ei, valid, w, ratio):
    score = (x @ w) / jnp.linalg.norm(w)
    N = x.shape[0]
    k = int(math.ceil(ratio * N))
    perm = jnp.argsort(-score)[:k]
    x_new = x[perm] * jnp.tanh(score[perm])[:, None]
    new_id = jnp.full((N,), -1, jnp.int32).at[perm].set(
        jnp.arange(k, dtype=jnp.int32))
    s2 = new_id[ei[0]]
    d2 = new_id[ei[1]]
    keep = ((s2 >= 0) & (d2 >= 0)).astype(x.dtype)
    v_new = valid * keep
    ei_new = jnp.stack([jnp.maximum(s2, 0), jnp.maximum(d2, 0)], axis=0)
    return x_new, ei_new, v_new, perm


def _knn_interpolate(x, pos_src, pos_tgt, k=3):
    d2 = (jnp.sum(pos_tgt ** 2, axis=1)[:, None]
          + jnp.sum(pos_src ** 2, axis=1)[None, :]
          - 2.0 * pos_tgt @ pos_src.T)
    neg, idx = jax.lax.top_k(-d2, k)
    d2k = jnp.maximum(-neg, 1e-16)
    w = 1.0 / d2k
    return jnp.sum(x[idx] * w[:, :, None], axis=1) / jnp.sum(w, axis=1)[:, None]


def _down(state, basis_by_K, p, Ka, Kb):
    x, ei, valid, pos, batch = state
    bw_a, bidx_a = basis_by_K[Ka]
    bw_b, bidx_b = basis_by_K[Kb]
    x = jax.nn.elu(_conv(x, ei[0], ei[1], valid, bw_a, bidx_a, p["conva"], Ka))
    x = jax.nn.elu(_conv(x, ei[0], ei[1], valid, bw_b, bidx_b, p["convb"], Kb))
    mean = jnp.mean(x, axis=0)
    var = jnp.var(x, axis=0)
    x = (x - mean) / jnp.sqrt(var + 1e-5) * p["bn_gamma"] + p["bn_beta"]
    back = (x, ei, valid, pos, batch)
    xn, ein, vn, perm = _topk_pool(x, ei, valid, p["pool_w"], 0.5)
    return (xn, ein, vn, pos[perm], batch[perm]), back


def _up(state, back, basis_by_K, conv_p, K):
    x, ei, valid, pos, batch = state
    if conv_p is not None:
        bw, bidx = basis_by_K[K]
        x = jax.nn.elu(_conv(x, ei[0], ei[1], valid, bw, bidx, conv_p, K))
    bx, bei, bvalid, bpos, bbatch = back
    x = _knn_interpolate(x, pos, bpos, 3)
    return (x, bei, bvalid, bpos, bbatch)


def kernel(x, edge_index, edge_attr, pos, batch, params):
    E = edge_index.shape[1]
    valid = jnp.ones((E,), x.dtype)
    basis_by_K = {K: _basis(edge_attr, K) for K in (3, 5)}
    basis_by_K[1] = (None, None)

    st = (x, edge_index, valid, pos, batch)
    st, b1 = _down(st, basis_by_K, params["down1"], 5, 5)
    st, b2 = _down(st, basis_by_K, params["down2"], 3, 3)
    pool2 = st
    st, b3 = _down(st, basis_by_K, params["down3"], 3, 3)
    pool3 = st
    st, b4 = _down(st, basis_by_K, params["down4"], 1, 1)

    bw3, bidx3 = basis_by_K[3]
    x4 = jax.nn.elu(_conv(st[0], st[1][0], st[1][1], st[2], bw3, bidx3,
                          params["score_fs"], 3))
    st = (x4,) + st[1:]
    st = _up(st, b4, basis_by_K, params["up1_conva"], 3)
    p3 = jax.nn.elu(_conv(pool3[0], pool3[1][0], pool3[1][1], pool3[2],
                          bw3, bidx3, params["score_pool3"], 3))
    st = (st[0] + p3,) + st[1:]
    st = _up(st, b3, basis_by_K, None, 3)
    p2 = jax.nn.elu(_conv(pool2[0], pool2[1][0], pool2[1][1], pool2[2],
                          bw3, bidx3, params["score_pool2"], 3))
    st = (st[0] + p2,) + st[1:]
    st = _up(st, b2, basis_by_K, None, 3)
    st = _up(st, b1, basis_by_K, params["up1_conva"], 3)

    bw5, bidx5 = basis_by_K[5]
    out = _conv(st[0], st[1][0], st[1][1], st[2], bw5, bidx5,
                params["convout"], 5)
    return out
